# baseline (speedup 1.0000x reference)
import functools

import jax
import jax.numpy as jnp
from jax import lax
from jax.experimental import pallas as pl
from jax.experimental.pallas import tpu as pltpu
from jax.experimental.pallas import tpu_sc as plsc

NC = 2
NS = 16
L = 16
ROWS = 32


def kernel(x, pe):
    batch, seq_len, d_model = x.shape
    nw = NC * NS
    seq_per_w = seq_len // nw
    n_steps = seq_per_w // ROWS
    mesh = plsc.VectorSubcoreMesh(core_axis_name="c", subcore_axis_name="s")

    @functools.partial(
        pl.kernel,
        mesh=mesh,
        out_type=jax.ShapeDtypeStruct((batch, seq_len, d_model), x.dtype),
        scratch_types=[
            pltpu.VMEM((ROWS, d_model), jnp.float32),
            pltpu.VMEM((ROWS, d_model), jnp.float32),
        ],
    )
    def sc_add(x_hbm, pe_hbm, out_hbm, pe_buf, x_buf):
        wid = lax.axis_index("s") * NC + lax.axis_index("c")
        base = wid * seq_per_w

        def step(i, carry):
            row0 = base + i * ROWS
            pltpu.sync_copy(pe_hbm.at[pl.ds(row0, ROWS), :], pe_buf)
            for b in range(batch):
                pltpu.sync_copy(x_hbm.at[b, pl.ds(row0, ROWS), :], x_buf)

                def row_loop(r, c2):
                    def vec_loop(j, c3):
                        sl = pl.ds(j * L, L)
                        x_buf[r, sl] = x_buf[r, sl] + pe_buf[r, sl]
                        return c3
                    return lax.fori_loop(0, d_model // L, vec_loop, c2)

                lax.fori_loop(0, ROWS, row_loop, 0)
                pltpu.sync_copy(x_buf, out_hbm.at[b, pl.ds(row0, ROWS), :])
            return carry

        lax.fori_loop(0, n_steps, step, 0)

    return sc_add(x, pe[:seq_len])


# speedup vs baseline: 2.8825x; 2.8825x over previous
import functools

import jax
import jax.numpy as jnp
from jax import lax
from jax.experimental import pallas as pl
from jax.experimental.pallas import tpu as pltpu
from jax.experimental.pallas import tpu_sc as plsc

NC = 2
NS = 16
L = 16
ROWS = 16


def kernel(x, pe):
    batch, seq_len, d_model = x.shape
    nw = NC * NS
    seq_per_w = seq_len // nw
    n_steps = seq_per_w // ROWS
    nj = d_model // L
    mesh = plsc.VectorSubcoreMesh(core_axis_name="c", subcore_axis_name="s")

    @functools.partial(
        pl.kernel,
        mesh=mesh,
        out_type=jax.ShapeDtypeStruct((batch, seq_len, d_model), x.dtype),
        scratch_types=[
            pltpu.VMEM((batch, ROWS, d_model), jnp.float32),
            pltpu.VMEM((2, ROWS, d_model), jnp.float32),
            [pltpu.SemaphoreType.DMA] * batch,
            [pltpu.SemaphoreType.DMA] * batch,
            [pltpu.SemaphoreType.DMA] * 2,
        ],
    )
    def sc_add(x_hbm, pe_hbm, out_hbm, x_bufs, pe_bufs, sx, so, spe):
        wid = lax.axis_index("s") * NC + lax.axis_index("c")
        base = wid * seq_per_w

        def x_copy(i, b):
            rows = pl.ds(base + i * ROWS, ROWS)
            return pltpu.make_async_copy(x_hbm.at[b, rows, :], x_bufs.at[b], sx[b])

        def out_copy(i, b):
            rows = pl.ds(base + i * ROWS, ROWS)
            return pltpu.make_async_copy(x_bufs.at[b], out_hbm.at[b, rows, :], so[b])

        def pe_copy(i, ph):
            rows = pl.ds(base + i * ROWS, ROWS)
            return pltpu.make_async_copy(pe_hbm.at[rows, :], pe_bufs.at[ph], spe[ph])

        def do_block(i, b, ph):
            if b < batch - 1:
                @pl.when(i > 0)
                def _():
                    out_copy(i - 1, b + 1).wait()
                x_copy(i, b + 1).start()
            else:
                @pl.when(i < n_steps - 1)
                def _():
                    out_copy(i, 0).wait()
                    x_copy(i + 1, 0).start()
            x_copy(i, b).wait()

            def row_loop(r, c):
                for j in range(nj):
                    sl = pl.ds(j * L, L)
                    plsc.addupdate(x_bufs.at[b, r, sl], pe_bufs[ph, r, sl])
                return c

            lax.fori_loop(0, ROWS, row_loop, 0)
            out_copy(i, b).start()

        def pair(k, c):
            for ph in range(2):
                i = k * 2 + ph

                @pl.when(i + 1 < n_steps)
                def _():
                    pe_copy(i + 1, 1 - ph).start()

                pe_copy(i, ph).wait()
                for b in range(batch):
                    do_block(i, b, ph)
            return c

        pe_copy(0, 0).start()
        x_copy(0, 0).start()
        lax.fori_loop(0, n_steps // 2, pair, 0)
        for b in range(batch):
            out_copy(n_steps - 1, b).wait()

    return sc_add(x, pe[:seq_len])


# speedup vs baseline: 4.6588x; 1.6162x over previous
import jax
import jax.numpy as jnp
from jax.experimental import pallas as pl
from jax.experimental.pallas import tpu as pltpu

S_BLK = 1024


def _add_body(x_ref, pe_ref, o_ref):
    o_ref[...] = x_ref[...] + pe_ref[...]


def kernel(x, pe):
    batch, seq_len, d_model = x.shape
    pe_used = pe[:seq_len]
    grid = (seq_len // S_BLK, batch)
    return pl.pallas_call(
        _add_body,
        grid=grid,
        in_specs=[
            pl.BlockSpec((1, S_BLK, d_model), lambda i, b: (b, i, 0)),
            pl.BlockSpec((S_BLK, d_model), lambda i, b: (i, 0)),
        ],
        out_specs=pl.BlockSpec((1, S_BLK, d_model), lambda i, b: (b, i, 0)),
        out_shape=jax.ShapeDtypeStruct((batch, seq_len, d_model), x.dtype),
        compiler_params=pltpu.CompilerParams(
            dimension_semantics=("arbitrary", "arbitrary"),
        ),
    )(x, pe_used)


# speedup vs baseline: 4.9598x; 1.0646x over previous
import jax
import jax.numpy as jnp
from jax.experimental import pallas as pl
from jax.experimental.pallas import tpu as pltpu

S_BLK = 2048


def _add_body(x_ref, pe_ref, o_ref):
    o_ref[...] = x_ref[...] + pe_ref[...]


def kernel(x, pe):
    batch, seq_len, d_model = x.shape
    pe_used = pe[:seq_len]
    grid = (seq_len // S_BLK, batch)
    return pl.pallas_call(
        _add_body,
        grid=grid,
        in_specs=[
            pl.BlockSpec((1, S_BLK, d_model), lambda i, b: (b, i, 0)),
            pl.BlockSpec((S_BLK, d_model), lambda i, b: (i, 0)),
        ],
        out_specs=pl.BlockSpec((1, S_BLK, d_model), lambda i, b: (b, i, 0)),
        out_shape=jax.ShapeDtypeStruct((batch, seq_len, d_model), x.dtype),
        compiler_params=pltpu.CompilerParams(
            dimension_semantics=("arbitrary", "arbitrary"),
        ),
    )(x, pe_used)
